# baseline (speedup 1.0000x reference)
import functools

import jax
import jax.numpy as jnp
from jax import lax
from jax.experimental import pallas as pl
from jax.experimental.pallas import tpu as pltpu
from jax.experimental.pallas import tpu_sc as plsc

NC = 2
NS = 16
NW = NC * NS
CHUNK = 128
NBUF = 4


@functools.partial(jax.jit, static_argnames=("nchunk", "d"))
def _sc_gather(idx, weight, *, nchunk, d):
  outer = nchunk // NBUF
  mesh = plsc.VectorSubcoreMesh(
      core_axis_name="c", subcore_axis_name="s",
      num_cores=NC, num_subcores=NS)

  @functools.partial(
      pl.kernel,
      out_type=jax.ShapeDtypeStruct((NW, nchunk, CHUNK, d), jnp.float32),
      mesh=mesh,
      scratch_types=[
          pltpu.VMEM((nchunk, CHUNK), jnp.int32),
          pltpu.VMEM((NBUF, CHUNK, d), jnp.float32),
          pltpu.SemaphoreType.DMA((NBUF,)),
          pltpu.SemaphoreType.DMA((NBUF,)),
      ],
      compiler_params=pltpu.CompilerParams(use_tc_tiling_on_sc=False),
  )
  def body(idx_hbm, table_hbm, out_hbm, idx_v, rows_v, sem_in, sem_out):
    wid = lax.axis_index("s") * NC + lax.axis_index("c")
    pltpu.sync_copy(idx_hbm.at[wid], idx_v)
    for b in range(NBUF):
      pltpu.async_copy(table_hbm.at[idx_v.at[b]], rows_v.at[b], sem_in.at[b])

    @pl.loop(0, outer)
    def _(g):
      for b in range(NBUF):
        j = g * NBUF + b
        pltpu.make_async_copy(
            table_hbm.at[idx_v.at[j]], rows_v.at[b], sem_in.at[b]).wait()
        pltpu.async_copy(rows_v.at[b], out_hbm.at[wid, j], sem_out.at[b])

        @pl.when(g < outer - 1)
        def _():
          pltpu.make_async_copy(
              rows_v.at[b], out_hbm.at[wid, j], sem_out.at[b]).wait()
          pltpu.async_copy(
              table_hbm.at[idx_v.at[j + NBUF]], rows_v.at[b], sem_in.at[b])

    for b in range(NBUF):
      j = (outer - 1) * NBUF + b
      pltpu.make_async_copy(
          rows_v.at[b], out_hbm.at[wid, j], sem_out.at[b]).wait()

  return body(idx, weight)


def kernel(inputs, weight):
  batch, hist = inputs.shape
  d = weight.shape[1]
  n = batch * hist
  assert n % (NW * CHUNK * NBUF) == 0
  nchunk = n // (NW * CHUNK)
  idx = inputs.reshape(NW, nchunk, CHUNK).astype(jnp.int32)
  out = _sc_gather(idx, weight, nchunk=nchunk, d=d)
  return out.reshape(batch, hist, d)
